# Initial kernel scaffold; baseline (speedup 1.0000x reference)
import jax
import jax.numpy as jnp
from jax.experimental import pallas as pl


def kernel(mask, color, params):
    raise NotImplementedError("write your pallas kernel here")


# speedup vs baseline: 589.7907x; 589.7907x over previous
import jax
import jax.numpy as jnp
import numpy as np
from jax.experimental import pallas as pl
from jax.experimental.pallas import tpu as pltpu

H = 512
W = 512
R = 32
GATE = 4.5


def _lrelu(x):
    return jnp.where(x >= 0, x, 0.01 * x)


def _bbr_kernel(xp_ref, sw_ref, sb_ref, b0w_ref, b0b_ref, b2w_ref, b2b_ref,
                f0w_ref, f0b_ref, f2w_ref, f2b_ref, s_ref, flag_ref):
    t = pl.program_id(0)
    r = t * R

    @pl.when(t == 0)
    def _():
        flag_ref[0, 0] = 0

    def mask_img(a, row0, col0):
        ri = row0 + jax.lax.broadcasted_iota(jnp.int32, a.shape, 0)
        ci = col0 + jax.lax.broadcasted_iota(jnp.int32, a.shape, 1)
        ok = (ri >= 0) & (ri < H) & (ci >= 0) & (ci < W)
        return jnp.where(ok, a, 0.0)

    xch = [xp_ref[c, pl.ds(r, R + 16), :] for c in range(3)]

    sp1 = jnp.full((R + 4, 516), sb_ref[0, 0], jnp.float32)
    for c in range(3):
        for ky in range(3):
            for kx in range(3):
                sp1 = sp1 + sw_ref[c, ky * 3 + kx] * xch[c][
                    5 + ky:5 + ky + R + 4, kx:kx + 516]
    sp1 = mask_img(sp1, r - 2, -2)
    sp2 = (4.0 * sp1[1:R + 3, 1:515] - sp1[0:R + 2, 1:515] - sp1[2:R + 4, 1:515]
           - sp1[1:R + 3, 0:514] - sp1[1:R + 3, 2:516])
    s_t = jax.nn.sigmoid(sp2)

    tacc = [jnp.full((R + 2, 514), b0b_ref[0, o], jnp.float32) for o in range(6)]
    for c in range(3):
        def xc(di, dj, c=c):
            return xch[c][7 + di:7 + di + R + 2, 2 + dj:2 + dj + 514]
        colc = (4.0 * xc(0, 0) - xc(-1, 0) - xc(1, 0) - xc(0, -1) - xc(0, 1))
        for o in range(6):
            tacc[o] = tacc[o] + b0w_ref[o, c] * colc
    t6 = [_lrelu(a) for a in tacc]
    wh1 = []
    for j in range(2):
        a = jnp.full((R + 2, 514), b2b_ref[0, j], jnp.float32)
        for o in range(6):
            a = a + b2w_ref[j, o] * t6[o]
        wh1.append(a)

    u = [mask_img(s_t, r - 1, -1),
         mask_img(wh1[0], r - 1, -1),
         mask_img(wh1[1], r - 1, -1)]
    v = []
    for o in range(6):
        a = jnp.full((R, 512), f0b_ref[0, o], jnp.float32)
        for c in range(3):
            for ky in range(3):
                for kx in range(3):
                    a = a + f0w_ref[o * 3 + c, ky * 3 + kx] * u[c][ky:ky + R, kx:kx + 512]
        v.append(_lrelu(a))
    wh = []
    for j in range(2):
        a = jnp.full((R, 512), f2b_ref[0, j], jnp.float32)
        for o in range(6):
            a = a + f2w_ref[j, o] * v[o]
        wh.append(a)

    s_ref[...] = s_t[1:R + 1, 1:513]
    hit = jnp.any((wh[0] > GATE) & (wh[1] > GATE))
    flag_ref[0, 0] = flag_ref[0, 0] | hit.astype(jnp.int32)


def _fill_kernel(s_ref, o_ref):
    k = pl.program_id(0)

    @pl.when(k == 0)
    def _():
        o_ref[0, :, :] = s_ref[...]

    @pl.when(k != 0)
    def _():
        o_ref[...] = jnp.zeros_like(o_ref)


def _fast_path(color, params):
    xp = jnp.pad(color[0], ((0, 0), (8, 8), (3, 3)))
    smem = pl.BlockSpec(memory_space=pltpu.SMEM)
    s, flag = pl.pallas_call(
        _bbr_kernel,
        grid=(H // R,),
        in_specs=[pl.BlockSpec((3, H + 16, W + 6), lambda i: (0, 0, 0))]
        + [smem] * 10,
        out_specs=[pl.BlockSpec((R, W), lambda i: (i, 0)), smem],
        out_shape=[jax.ShapeDtypeStruct((H, W), jnp.float32),
                   jax.ShapeDtypeStruct((1, 1), jnp.int32)],
    )(
        xp,
        params['score_w'][0].reshape(3, 9),
        params['score_b'].reshape(1, 1),
        params['bbx0_w'][:, :, 0, 0],
        params['bbx0_b'].reshape(1, 6),
        params['bbx2_w'][:, :, 0, 0],
        params['bbx2_b'].reshape(1, 2),
        params['ft0_w'].reshape(18, 9),
        params['ft0_b'].reshape(1, 6),
        params['ft2_w'][:, :, 0, 0],
        params['ft2_b'].reshape(1, 2),
    )
    out = pl.pallas_call(
        _fill_kernel,
        grid=(101,),
        in_specs=[pl.BlockSpec((H, W), lambda k: (0, 0))],
        out_specs=pl.BlockSpec((1, H, W), lambda k: (k, 0, 0)),
        out_shape=jax.ShapeDtypeStruct((101, H, W), jnp.float32),
    )(s)
    return out[None], flag


_LAP = jnp.array([[0., -1., 0.], [-1., 4., -1.], [0., -1., 0.]],
                 dtype=jnp.float32).reshape(1, 1, 3, 3)


def _conv(x, w, b=None, stride=1, padding=0, groups=1):
    y = jax.lax.conv_general_dilated(x, w, window_strides=(stride, stride),
                                     padding=((padding, padding), (padding, padding)),
                                     dimension_numbers=('NCHW', 'OIHW', 'NCHW'),
                                     feature_group_count=groups)
    if b is not None:
        y = y + b[None, :, None, None]
    return y


def _bbr(p, x):
    C = x.shape[1]
    lapC = jnp.tile(_LAP, (C, 1, 1, 1))
    col = _conv(x, lapC, None, 1, 1, groups=C)
    wh = _conv(_lrelu(_conv(col, p['bbx0_w'], p['bbx0_b'])), p['bbx2_w'], p['bbx2_b'])
    s = jax.nn.sigmoid(_conv(_conv(x, p['score_w'], p['score_b'], 1, 1), _LAP, None, 1, 1))
    wh = _conv(_lrelu(_conv(jnp.concatenate([s, wh], axis=1), p['ft0_w'], p['ft0_b'], 1, 1)), p['ft2_w'], p['ft2_b'])
    return jnp.concatenate([s, wh], axis=1)


def _cls_net(p, m, c):
    m = _lrelu(_conv(m, p['bs0_w'], p['bs0_b'], 3))
    m = _conv(m, p['bs2_w'], p['bs2_b'])
    m = _lrelu(_conv(m, p['bs3_w'], p['bs3_b'], 3))
    m = _conv(m, p['bs5_w'], p['bs5_b'])
    m = _lrelu(_conv(m, p['bs6_w'], p['bs6_b'], 4))
    m = _conv(m, p['bs8_w'], p['bs8_b'])
    m = _lrelu(_conv(m, p['bs9_w'], p['bs9_b'], 5))
    m = _conv(m, p['bs11_w'], p['bs11_b'])
    c = _lrelu(_conv(c, p['cs0_w'], p['cs0_b']))
    c = _conv(c, p['cs2_w'], p['cs2_b'])
    c = _lrelu(_conv(c, p['cs3_w'], p['cs3_b'], 9))
    c = _conv(c, p['cs5_w'], p['cs5_b'])
    c = _lrelu(_conv(c, p['cs6_w'], p['cs6_b'], 4))
    c = _conv(c, p['cs8_w'], p['cs8_b'])
    c = _lrelu(_conv(c, p['cs9_w'], p['cs9_b'], 5))
    c = _conv(c, p['cs11_w'], p['cs11_b'])
    z = _lrelu(_conv(jnp.concatenate([m, c], axis=1), p['cl0_w'], p['cl0_b']))
    return _conv(z, p['cl2_w'], p['cl2_b'])


def _resize_idx(n_in, n_out=900):
    return np.clip(np.floor(np.arange(n_out) * (n_in / max(n_out, 1))).astype(np.int64),
                   0, max(n_in - 1, 0))


def _slice_bounds(i, j, n):
    i = jnp.where(i < 0, i + n, i)
    j = jnp.where(j < 0, j + n, j)
    i = jnp.clip(i, 0, n)
    j = jnp.clip(j, 0, n)
    return i, jnp.maximum(j - i, 0)


def _extract(a, box, table):
    _, _, h, w = a.shape
    x1, y1, x2, y2 = box[0], box[1], box[2], box[3]
    sy, ny = _slice_bounds(y1, y2, h)
    sx, nx = _slice_bounds(x1, x2, w)
    yi = sy + table[ny]
    xi = sx + table[nx]
    return a[:, :, yi, :][:, :, :, xi]


def _select(bbx_out):
    B, _, h, w = bbx_out.shape
    sf = bbx_out[:, 0].reshape(B, -1)
    idx = jnp.argsort(-sf, axis=1, stable=True)[:, :10]
    sel = jnp.zeros(sf.shape, dtype=bool).at[jnp.arange(B)[:, None], idx].set(True)
    sel = sel | (sf > 0.8)
    wf = bbx_out[:, 1].reshape(B, -1)
    hf = bbx_out[:, 2].reshape(B, -1)
    sel = sel & (wf > 5) & (hf > 5)
    yy = jnp.broadcast_to(jnp.arange(h, dtype=jnp.float32)[:, None], (h, w)).reshape(-1)
    xx = jnp.broadcast_to(jnp.arange(w, dtype=jnp.float32)[None, :], (h, w)).reshape(-1)
    yy = jnp.broadcast_to(yy[None], (B, h * w))
    xx = jnp.broadcast_to(xx[None], (B, h * w))
    ws = wf
    x1 = jnp.floor(xx - ws).astype(jnp.int32)
    x2 = jnp.ceil(xx + ws).astype(jnp.int32)
    y1 = jnp.floor(yy - ws).astype(jnp.int32)
    y2 = jnp.ceil(yy + ws).astype(jnp.int32)
    boxes = jnp.stack([x1, y1, x2, y2], axis=-1).reshape(-1, 4)
    return sel, boxes


def _slow_path(mask, color, params, num_classes=100):
    bbx_out = _bbr(params, color)
    sel, boxes = _select(bbx_out)
    B, _, h, w = color.shape
    score = bbx_out[:, 0:1]
    table = jnp.asarray(np.stack([_resize_idx(n) for n in range(max(h, w) + 1)]).astype(np.int32))

    def body(carry, xj):
        s, box = xj

        def on(_):
            m = _extract(mask, box, table)
            c = _extract(color, box, table)
            return _cls_net(params, m, c).reshape(num_classes)

        def off(_):
            return jnp.zeros((num_classes,), jnp.float32)

        val = jax.lax.cond(s, on, off, None)
        return carry, val

    _, vals = jax.lax.scan(body, 0, (sel.reshape(-1), boxes))
    cls = vals.reshape(B, num_classes, h, w)
    return jnp.concatenate([score, cls], axis=1)


def kernel(mask, color, params):
    fast_out, flag = _fast_path(color, params)
    return jax.lax.cond(flag[0, 0] > 0,
                        lambda _: _slow_path(mask, color, params),
                        lambda _: fast_out,
                        None)
